# Initial kernel scaffold; baseline (speedup 1.0000x reference)
import jax
import jax.numpy as jnp
from jax.experimental import pallas as pl


def kernel(target_feat, residue_index, msa_feat, w_zsum, b_zsum, w_tfm, b_tfm, w_rel, w_msa, b_msa):
    raise NotImplementedError("write your pallas kernel here")


# speedup vs baseline: 1.2470x; 1.2470x over previous
import functools

import jax
import jax.numpy as jnp
from jax import lax
from jax.experimental import pallas as pl
from jax.experimental.pallas import tpu as pltpu


def _fused_kernel(tf_ref, ri_row_ref, ri_col_ref, msa_ref,
                  w_zsum_ref, b_zsum_ref, w_tfm_ref, b_tfm_ref,
                  w_rel_ref, w_msa_ref, b_msa_ref,
                  msa_out_ref, pair_out_ref, *, relpos_k):
    ts, n, msa_dim = msa_ref.shape[1], msa_ref.shape[2], msa_ref.shape[3]
    c_m = w_msa_ref.shape[1]
    num_bins, c_z = w_rel_ref.shape
    ti = ri_row_ref.shape[1]

    tf = tf_ref[0]

    tf_m = jnp.dot(tf, w_tfm_ref[...],
                   preferred_element_type=jnp.float32) + b_tfm_ref[...]
    msa = msa_ref[0].reshape(ts * n, msa_dim).astype(jnp.bfloat16)
    l1 = jnp.dot(msa, w_msa_ref[...].astype(jnp.bfloat16),
                 preferred_element_type=jnp.float32) + b_msa_ref[...]
    msa_out_ref[0] = (l1.reshape(ts, n, c_m)
                      + tf_m[None, :, :]).astype(msa_out_ref.dtype)

    bias = jnp.dot(tf, w_zsum_ref[...],
                   preferred_element_type=jnp.float32) + b_zsum_ref[...]
    ri_i = ri_row_ref[0]
    ri_j = ri_col_ref[0]
    idx = jnp.clip(ri_i - ri_j + relpos_k, 0, num_bins - 1)
    lane = lax.broadcasted_iota(jnp.int32, (ti, n, num_bins), 2)
    one_hot = (lane == idx[:, :, None]).astype(jnp.bfloat16)
    relpos = jnp.dot(one_hot.reshape(ti * n, num_bins),
                     w_rel_ref[...].astype(jnp.bfloat16),
                     preferred_element_type=jnp.float32)
    pair_out_ref[0] = (relpos.reshape(ti, n, c_z)
                       + bias[None, :, :]).astype(pair_out_ref.dtype)


def _pick_steps(S, N):
    for g in (16, 8, 4, 2):
        if S % g == 0 and N % g == 0 and (N // g) % 8 == 0:
            return g
    return 1


def kernel(target_feat, residue_index, msa_feat, w_zsum, b_zsum, w_tfm, b_tfm,
           w_rel, w_msa, b_msa):
    B, N, tf_dim = target_feat.shape
    S, msa_dim = msa_feat.shape[1], msa_feat.shape[3]
    num_bins, c_z = w_rel.shape
    c_m = w_msa.shape[1]
    relpos_k = (num_bins - 1) // 2

    g = _pick_steps(S, N)
    ts, ti = S // g, N // g

    ri = residue_index.astype(jnp.int32)
    ri_row = ri.reshape(B, N, 1)
    ri_col = ri.reshape(B, 1, N)

    body = functools.partial(_fused_kernel, relpos_k=relpos_k)
    msa_out, pair_out = pl.pallas_call(
        body,
        out_shape=(jax.ShapeDtypeStruct((B, S, N, c_m), jnp.float32),
                   jax.ShapeDtypeStruct((B, N, N, c_z), jnp.float32)),
        grid=(B, g),
        in_specs=[
            pl.BlockSpec((1, N, tf_dim), lambda b, s: (b, 0, 0)),
            pl.BlockSpec((1, ti, 1), lambda b, s: (b, s, 0)),
            pl.BlockSpec((1, 1, N), lambda b, s: (b, 0, 0)),
            pl.BlockSpec((1, ts, N, msa_dim), lambda b, s: (b, s, 0, 0)),
            pl.BlockSpec((tf_dim, c_z), lambda b, s: (0, 0)),
            pl.BlockSpec((1, c_z), lambda b, s: (0, 0)),
            pl.BlockSpec((tf_dim, c_m), lambda b, s: (0, 0)),
            pl.BlockSpec((1, c_m), lambda b, s: (0, 0)),
            pl.BlockSpec((num_bins, c_z), lambda b, s: (0, 0)),
            pl.BlockSpec((msa_dim, c_m), lambda b, s: (0, 0)),
            pl.BlockSpec((1, c_m), lambda b, s: (0, 0)),
        ],
        out_specs=(pl.BlockSpec((1, ts, N, c_m), lambda b, s: (b, s, 0, 0)),
                   pl.BlockSpec((1, ti, N, c_z), lambda b, s: (b, s, 0, 0))),
        compiler_params=pltpu.CompilerParams(
            dimension_semantics=("parallel", "parallel"),
            vmem_limit_bytes=48 * 1024 * 1024),
    )(target_feat, ri_row, ri_col, msa_feat,
      w_zsum, b_zsum, w_tfm, b_tfm, w_rel, w_msa, b_msa)
    return msa_out, pair_out
